# baseline (speedup 1.0000x reference)
import functools

import jax
import jax.numpy as jnp
from jax import lax
from jax.experimental import pallas as pl
from jax.experimental.pallas import tpu as pltpu
from jax.experimental.pallas import tpu_sc as plsc

_INFO = plsc.get_sparse_core_info()
_NC, _NS = _INFO.num_cores, _INFO.num_subcores
_NW = _NC * _NS

_NBUF = 4


def _embed_lookup(idx3, table, b, s, d):
    mesh = plsc.VectorSubcoreMesh(core_axis_name="c", subcore_axis_name="s")
    rows_per_w = b // _NW
    s0 = min(s, 128)
    s1 = s - s0

    @functools.partial(
        pl.kernel,
        out_type=jax.ShapeDtypeStruct((b, s, d), jnp.float32),
        mesh=mesh,
        compiler_params=pltpu.CompilerParams(use_tc_tiling_on_sc=False),
        scratch_types=[
            pltpu.VMEM((rows_per_w, s), jnp.int32),
            pltpu.VMEM((_NBUF, s, d), jnp.float32),
            pltpu.SemaphoreType.DMA((_NBUF,)),
            pltpu.SemaphoreType.DMA((_NBUF,)),
        ],
    )
    def k(idx_hbm, table_hbm, out_hbm, idx_v, rows_v, gsem, wsem):
        wid = lax.axis_index("s") * _NC + lax.axis_index("c")
        base = wid * rows_per_w
        pltpu.sync_copy(idx_hbm.at[wid], idx_v)

        def gathers(r, buf):
            g0 = pltpu.make_async_copy(
                table_hbm.at[idx_v.at[r, pl.ds(0, s0)]],
                rows_v.at[buf, pl.ds(0, s0)], gsem.at[buf])
            if s1 == 0:
                return (g0,)
            g1 = pltpu.make_async_copy(
                table_hbm.at[idx_v.at[r, pl.ds(s0, s1)]],
                rows_v.at[buf, pl.ds(s0, s1)], gsem.at[buf])
            return (g0, g1)

        def write(r, buf):
            return pltpu.make_async_copy(
                rows_v.at[buf], out_hbm.at[base + r], wsem.at[buf])

        def start_gathers(r, buf):
            for g in gathers(r, buf):
                g.start()

        def wait_gathers(r, buf):
            for g in gathers(r, buf):
                g.wait()

        for buf in range(_NBUF):
            start_gathers(buf, buf)

        n_groups = rows_per_w // _NBUF

        def body(g, carry):
            for buf in range(_NBUF):
                r = g * _NBUF + buf
                wait_gathers(r, buf)
                write(r, buf).start()
            for buf in range(_NBUF):
                r = g * _NBUF + buf
                write(r, buf).wait()
                start_gathers(r + _NBUF, buf)
            return carry

        lax.fori_loop(0, n_groups - 1, body, 0)

        g = n_groups - 1
        for buf in range(_NBUF):
            r = g * _NBUF + buf
            wait_gathers(r, buf)
            write(r, buf).start()
        for buf in range(_NBUF):
            write(g * _NBUF + buf, buf).wait()

    return k(idx3, table)


def kernel(input_, weight):
    b, s = input_.shape
    d = weight.shape[1]
    assert b % (_NW * _NBUF) == 0
    idx3 = input_.reshape(_NW, b // _NW, s).astype(jnp.int32)
    return _embed_lookup(idx3, weight, b, s, d)
